# baseline (speedup 1.0000x reference)
import functools

import jax
import jax.numpy as jnp
from jax import lax
from jax.experimental import pallas as pl
from jax.experimental.pallas import tpu as pltpu
from jax.experimental.pallas import tpu_sc as plsc

NC, NS, L = 2, 16, 16
D, H, W = 10, 26, 26
P = D * H * W
NFULL = P // L
B, COUT = 8, 128
CPW = COUT // (NC * NS)


def _pos_body(row_hbm, col_hbm, depth_hbm, out_hbm, tbl26, tbl10, r0, r1, r2, r3, sem):
    wid = lax.axis_index("s") * NC + lax.axis_index("c")
    c0 = wid * CPW
    rows = (r0, r1, r2, r3)

    is_a = wid <= 10
    is_b = (wid >= 11) & (wid <= 21)
    is_c = wid >= 22

    @pl.when(is_a)
    def _():
        pltpu.sync_copy(col_hbm, tbl26)

    @pl.when(is_b)
    def _():
        pltpu.sync_copy(row_hbm, tbl26)

    @pl.when(is_c)
    def _():
        pltpu.sync_copy(depth_hbm, tbl10)

    def build_row(rbuf, tbl, j, pat_fn, col_off):
        col = jnp.full((L,), c0 + j - col_off, jnp.int32)

        def blk(t, carry):
            base = t * L
            pvec = lax.iota(jnp.int32, L) + base
            idx = pat_fn(pvec) * 44 + col
            rbuf[pl.ds(base, L)] = plsc.load_gather(tbl, [idx])
            return carry

        lax.fori_loop(0, NFULL, blk, 0)
        pvec = lax.iota(jnp.int32, L) + (P - L)
        idx = pat_fn(pvec) * 44 + col
        rbuf[pl.ds(P - L, L)] = plsc.load_gather(tbl, [idx])

    def run(tbl, pat_fn, col_off):
        copies = []
        for j in range(CPW):
            build_row(rows[j], tbl, j, pat_fn, col_off)
            for b in range(B):
                dst = out_hbm.at[pl.ds((b * COUT + c0 + j) * P, P)]
                copies.append(pltpu.async_copy(rows[j], dst, sem))
        for cp in copies:
            cp.wait()

    @pl.when(is_a)
    def _():
        run(tbl26, lambda p: lax.rem(p, W), 0)

    @pl.when(is_b)
    def _():
        run(tbl26, lambda p: lax.rem(lax.div(p, W), H), 44)

    @pl.when(is_c)
    def _():
        run(tbl10, lambda p: jnp.minimum(lax.div(p, H * W), D - 1), 88)


@jax.jit
def _pos_embed(row_w, col_w, depth_w):
    mesh = plsc.VectorSubcoreMesh(
        core_axis_name="c", subcore_axis_name="s", num_cores=NC, num_subcores=NS
    )
    k = pl.kernel(
        _pos_body,
        out_type=jax.ShapeDtypeStruct((B * COUT * P,), jnp.float32),
        mesh=mesh,
        compiler_params=pltpu.CompilerParams(needs_layout_passes=False),
        scratch_types=[
            pltpu.VMEM((H * 44,), jnp.float32),
            pltpu.VMEM((D * 44,), jnp.float32),
            pltpu.VMEM((P,), jnp.float32),
            pltpu.VMEM((P,), jnp.float32),
            pltpu.VMEM((P,), jnp.float32),
            pltpu.VMEM((P,), jnp.float32),
            pltpu.SemaphoreType.DMA,
        ],
    )
    return k(row_w.reshape(-1), col_w.reshape(-1), depth_w.reshape(-1))


def kernel(x, row_w, col_w, depth_w):
    out = _pos_embed(row_w, col_w, depth_w)
    return out.reshape(B, COUT, D, H, W)


# speedup vs baseline: 3.8237x; 3.8237x over previous
import jax
import jax.numpy as jnp
from jax import lax
from jax.experimental import pallas as pl
from jax.experimental.pallas import tpu as pltpu
from jax.experimental.pallas import tpu_sc as plsc

NC, NS, L = 2, 16, 16
D, H, W = 10, 26, 26
P = D * H * W
B, COUT = 8, 128
PPW = 212
NBLK = COUT // L

ROW_BASE = H * 44
DEP_BASE = 2 * H * 44
TBL_LEN = DEP_BASE + D * 44


def _pos_body(row_hbm, col_hbm, depth_hbm, out_hbm, tbl, src, sem):
    wid = lax.axis_index("s") * NC + lax.axis_index("c")
    p0 = jnp.minimum(wid * PPW, P - PPW)

    pltpu.sync_copy(col_hbm, tbl.at[pl.ds(0, H * 44)])
    pltpu.sync_copy(row_hbm, tbl.at[pl.ds(ROW_BASE, H * 44)])
    pltpu.sync_copy(depth_hbm, tbl.at[pl.ds(DEP_BASE, D * 44)])

    def row_body(r, carry):
        p = p0 + r
        i = lax.rem(p, W)
        j = lax.rem(lax.div(p, W), H)
        k = lax.div(p, H * W)
        oi = 44 * i
        oj = ROW_BASE - 44 + 44 * j
        ok = DEP_BASE - 88 + 44 * k
        for m in range(NBLK):
            c = lax.iota(jnp.int32, L) + (L * m)
            if m < 2:
                off = jnp.full((L,), oi, jnp.int32)
            elif m == 2:
                off = jnp.where(c < 44, oi, oj)
            elif m < 5:
                off = jnp.full((L,), oj, jnp.int32)
            elif m == 5:
                off = jnp.where(c < 88, oj, ok)
            else:
                off = jnp.full((L,), ok, jnp.int32)
            src[r, pl.ds(L * m, L)] = plsc.load_gather(tbl, [c + off])
        return carry

    lax.fori_loop(0, PPW, row_body, 0)

    copies = [
        pltpu.async_copy(src, out_hbm.at[pl.ds(p0, PPW), b], sem)
        for b in range(B)
    ]
    for cp in copies:
        cp.wait()


@jax.jit
def _pos_embed(row_w, col_w, depth_w):
    mesh = plsc.VectorSubcoreMesh(
        core_axis_name="c", subcore_axis_name="s", num_cores=NC, num_subcores=NS
    )
    k = pl.kernel(
        _pos_body,
        out_type=jax.ShapeDtypeStruct((P, B, COUT), jnp.float32),
        mesh=mesh,
        compiler_params=pltpu.CompilerParams(needs_layout_passes=False),
        scratch_types=[
            pltpu.VMEM((TBL_LEN,), jnp.float32),
            pltpu.VMEM((PPW, COUT), jnp.float32),
            pltpu.SemaphoreType.DMA,
        ],
    )
    return k(row_w.reshape(-1), col_w.reshape(-1), depth_w.reshape(-1))


def kernel(x, row_w, col_w, depth_w):
    out = _pos_embed(row_w, col_w, depth_w)
    return out.reshape(D, H, W, B, COUT).transpose(3, 4, 0, 1, 2)


# speedup vs baseline: 4.6814x; 1.2243x over previous
import jax
import jax.numpy as jnp
from jax import lax
from jax.experimental import pallas as pl
from jax.experimental.pallas import tpu as pltpu
from jax.experimental.pallas import tpu_sc as plsc

NC, NS, L = 2, 16, 16
D, H, W = 10, 26, 26
P = D * H * W
B, COUT = 8, 128
PPW = 212
CHUNKS = (52, 52, 52, 56)
NBLK = COUT // L

ROW_BASE = H * 44
DEP_BASE = 2 * H * 44
TBL_LEN = DEP_BASE + D * 44


def _pos_body(tbl_hbm, out_hbm, tbl, src, sem):
    wid = lax.axis_index("s") * NC + lax.axis_index("c")
    p0 = jnp.minimum(wid * PPW, P - PPW)

    pltpu.sync_copy(tbl_hbm, tbl)

    def row_body(r, carry):
        p = p0 + r
        i = lax.rem(p, W)
        j = lax.rem(lax.div(p, W), H)
        k = lax.div(p, H * W)
        oi = 44 * i
        oj = ROW_BASE - 44 + 44 * j
        ok = DEP_BASE - 88 + 44 * k
        for m in range(NBLK):
            c = lax.iota(jnp.int32, L) + (L * m)
            if m < 2:
                off = jnp.full((L,), oi, jnp.int32)
            elif m == 2:
                off = jnp.where(c < 44, oi, oj)
            elif m < 5:
                off = jnp.full((L,), oj, jnp.int32)
            elif m == 5:
                off = jnp.where(c < 88, oj, ok)
            else:
                off = jnp.full((L,), ok, jnp.int32)
            src[r, pl.ds(L * m, L)] = plsc.load_gather(tbl, [c + off])
        return carry

    copies = []
    base = 0
    for cnt in CHUNKS:
        lax.fori_loop(base, base + cnt, row_body, 0)
        for b in range(B):
            copies.append(
                pltpu.async_copy(
                    src.at[pl.ds(base, cnt)],
                    out_hbm.at[pl.ds(p0 + base, cnt), b],
                    sem,
                )
            )
        base += cnt
    for cp in copies:
        cp.wait()


@jax.jit
def _pos_embed(row_w, col_w, depth_w):
    mesh = plsc.VectorSubcoreMesh(
        core_axis_name="c", subcore_axis_name="s", num_cores=NC, num_subcores=NS
    )
    k = pl.kernel(
        _pos_body,
        out_type=jax.ShapeDtypeStruct((P, B, COUT), jnp.float32),
        mesh=mesh,
        compiler_params=pltpu.CompilerParams(needs_layout_passes=False),
        scratch_types=[
            pltpu.VMEM((TBL_LEN,), jnp.float32),
            pltpu.VMEM((PPW, COUT), jnp.float32),
            pltpu.SemaphoreType.DMA,
        ],
    )
    cat = jnp.concatenate(
        [col_w.reshape(-1), row_w.reshape(-1), depth_w.reshape(-1)]
    )
    return k(cat)


def kernel(x, row_w, col_w, depth_w):
    out = _pos_embed(row_w, col_w, depth_w)
    return out.reshape(D, H, W, B, COUT).transpose(3, 4, 0, 1, 2)
